# baseline (speedup 1.0000x reference)
import functools
import math

import jax
import jax.numpy as jnp
from jax import lax
from jax.experimental import pallas as pl
from jax.experimental.pallas import tpu as pltpu
from jax.experimental.pallas import tpu_sc as plsc

D_MODEL = 64
SCALE = math.sqrt(D_MODEL)
V = 1000000
NW, NJ, CH = 32, 200, 128


def _emb_kernel(x_hbm, tab_hbm, out_hbm, x_v, g0, o0, gsem, osem):
    wid = lax.axis_index("s") * 2 + lax.axis_index("c")
    pltpu.sync_copy(x_hbm.at[wid], x_v)

    lane = lax.broadcasted_iota(jnp.int32, (16,), 0)
    gvecs = [(16 * fb + lane) // 8 for fb in range(4)]
    rvecs = [(16 * fb + lane) % 8 for fb in range(4)]

    def body_j(j, c):
        pltpu.async_copy(tab_hbm.at[x_v.at[j]], g0, gsem).wait()

        @plsc.parallel_loop(0, CH, unroll=2)
        def row(l):
            l16 = jnp.full((16,), 0, jnp.int32) + l
            for fb in range(4):
                v = g0[l, pl.ds(16 * fb, 16)]
                plsc.store_scatter(o0, [gvecs[fb], rvecs[fb], l16], v * SCALE)

        pltpu.sync_copy(o0, out_hbm.at[j, :, wid])
        return c

    lax.fori_loop(0, NJ, body_j, 0)


@jax.jit
def _embedding(xw, tabp):
    mesh = plsc.VectorSubcoreMesh(core_axis_name="c", subcore_axis_name="s")
    kfn = functools.partial(
        pl.kernel, mesh=mesh,
        out_type=jax.ShapeDtypeStruct((NJ, 8, NW, 8, 128), jnp.float32),
        scratch_types=[
            pltpu.VMEM((NJ, CH), jnp.int32),
            pltpu.VMEM((CH, 128), jnp.float32),
            pltpu.VMEM((8, 8, 128), jnp.float32),
            pltpu.SemaphoreType.DMA,
            pltpu.SemaphoreType.DMA,
        ],
        compiler_params=pltpu.CompilerParams(
            use_tc_tiling_on_sc=True, needs_layout_passes=False
        ),
    )(_emb_kernel)
    return kfn(xw, tabp)


def kernel(x, table):
    xw = x.astype(jnp.int32).T.reshape(NJ, NW, CH).transpose(1, 0, 2)
    tabp = jnp.pad(table, ((0, 0), (0, 64)))
    out5 = _embedding(xw, tabp)
    out = out5.transpose(2, 4, 0, 1, 3).reshape(NW * CH, NJ, D_MODEL)
    return out


# speedup vs baseline: 1.5842x; 1.5842x over previous
import functools
import math

import jax
import jax.numpy as jnp
from jax import lax
from jax.experimental import pallas as pl
from jax.experimental.pallas import tpu as pltpu
from jax.experimental.pallas import tpu_sc as plsc

D_MODEL = 64
SCALE = math.sqrt(D_MODEL)

V = 1000000
NC = 2
NS = 16
NW = NC * NS

B_TOTAL = 4096 * 200
B_PER_W = B_TOTAL // NW
CH = 128
NJ = B_PER_W // CH


def _emb_kernel(x_hbm, tab_hbm, out_hbm,
                x_v, g0, g1, s0, s1, gsem0, gsem1, osem0, osem1):
    wid = lax.axis_index("s") * NC + lax.axis_index("c")
    base = wid * B_PER_W
    pltpu.sync_copy(x_hbm.at[wid], x_v)

    gbufs, sbufs = (g0, g1), (s0, s1)
    gsems, osems = (gsem0, gsem1), (osem0, osem1)

    def start_gather(j, b):
        pltpu.async_copy(tab_hbm.at[x_v.at[j]], gbufs[b], gsems[b])

    def wait_gather(b):
        pltpu.make_async_copy(tab_hbm.at[x_v.at[0]], gbufs[b],
                              gsems[b]).wait()

    def scale(b):
        @plsc.parallel_loop(0, CH, unroll=2)
        def row(l):
            for fb in range(D_MODEL // 16):
                v = gbufs[b][l, pl.ds(16 * fb, 16)]
                sbufs[b][l, pl.ds(16 * fb, 16)] = v * SCALE

    def start_store(j, b):
        pltpu.async_copy(sbufs[b], out_hbm.at[pl.ds(base + j * CH, CH)],
                         osems[b])

    def wait_store(b):
        pltpu.make_async_copy(sbufs[b], out_hbm.at[pl.ds(base, CH)],
                              osems[b]).wait()

    start_gather(0, 0)
    start_gather(1, 1)
    for b in range(2):
        wait_gather(b)
        scale(b)
        start_store(b, b)
        start_gather(b + 2, b)

    def body(i, c):
        for b in range(2):
            j = 2 * i + b
            wait_gather(b)
            wait_store(b)
            scale(b)
            start_store(j, b)

            @pl.when(j + 2 < NJ)
            def _():
                start_gather(j + 2, b)
        return c

    lax.fori_loop(1, NJ // 2, body, 0)
    wait_store(0)
    wait_store(1)


@jax.jit
def _embedding(xw, tabp):
    mesh = plsc.VectorSubcoreMesh(core_axis_name="c", subcore_axis_name="s")
    kfn = functools.partial(
        pl.kernel,
        mesh=mesh,
        out_type=jax.ShapeDtypeStruct((B_TOTAL, D_MODEL), jnp.float32),
        scratch_types=[
            pltpu.VMEM((NJ, CH), jnp.int32),
            pltpu.VMEM((CH, 128), jnp.float32),
            pltpu.VMEM((CH, 128), jnp.float32),
            pltpu.VMEM((CH, D_MODEL), jnp.float32),
            pltpu.VMEM((CH, D_MODEL), jnp.float32),
            pltpu.SemaphoreType.DMA,
            pltpu.SemaphoreType.DMA,
            pltpu.SemaphoreType.DMA,
            pltpu.SemaphoreType.DMA,
        ],
        compiler_params=pltpu.CompilerParams(
            use_tc_tiling_on_sc=True, needs_layout_passes=False
        ),
    )(_emb_kernel)
    return kfn(xw, tabp)


def kernel(x, table):
    xw = x.astype(jnp.int32).reshape(NW, NJ, CH)
    tabp = jnp.pad(table, ((0, 0), (0, D_MODEL)))
    out = _embedding(xw, tabp)
    return out.reshape(x.shape[0], x.shape[1], D_MODEL)


# speedup vs baseline: 1.5855x; 1.0008x over previous
import functools
import math

import jax
import jax.numpy as jnp
from jax import lax
from jax.experimental import pallas as pl
from jax.experimental.pallas import tpu as pltpu
from jax.experimental.pallas import tpu_sc as plsc

D_MODEL = 64
SCALE = math.sqrt(D_MODEL)

V = 1000000
NC = 2
NS = 16
NW = NC * NS

B_TOTAL = 4096 * 200
B_PER_W = B_TOTAL // NW
CH = 128
NJ = B_PER_W // CH


def _emb_kernel(x_hbm, tab_hbm, out_hbm,
                x_v, g0, g1, s0, s1, gsem0, gsem1, osem0, osem1):
    wid = lax.axis_index("s") * NC + lax.axis_index("c")
    base = wid * B_PER_W
    pltpu.sync_copy(x_hbm.at[wid], x_v)

    gbufs, sbufs = (g0, g1), (s0, s1)
    gsems, osems = (gsem0, gsem1), (osem0, osem1)

    def start_gather(j, b):
        pltpu.async_copy(tab_hbm.at[x_v.at[j]], gbufs[b], gsems[b])

    def wait_gather(b):
        pltpu.make_async_copy(tab_hbm.at[x_v.at[0]], gbufs[b],
                              gsems[b]).wait()

    def scale(b):
        @plsc.parallel_loop(0, CH, unroll=1)
        def row(l):
            for fb in range(D_MODEL // 16):
                v = gbufs[b][l, pl.ds(16 * fb, 16)]
                sbufs[b][l, pl.ds(16 * fb, 16)] = v * SCALE

    def start_store(j, b):
        pltpu.async_copy(sbufs[b], out_hbm.at[pl.ds(base + j * CH, CH)],
                         osems[b])

    def wait_store(b):
        pltpu.make_async_copy(sbufs[b], out_hbm.at[pl.ds(base, CH)],
                              osems[b]).wait()

    start_gather(0, 0)
    start_gather(1, 1)
    for b in range(2):
        wait_gather(b)
        scale(b)
        start_store(b, b)
        start_gather(b + 2, b)

    def body(i, c):
        for b in range(2):
            j = 2 * i + b
            wait_gather(b)
            wait_store(b)
            scale(b)
            start_store(j, b)

            @pl.when(j + 2 < NJ)
            def _():
                start_gather(j + 2, b)
        return c

    lax.fori_loop(1, NJ // 2, body, 0)
    wait_store(0)
    wait_store(1)


@jax.jit
def _embedding(xw, tabp):
    mesh = plsc.VectorSubcoreMesh(core_axis_name="c", subcore_axis_name="s")
    kfn = functools.partial(
        pl.kernel,
        mesh=mesh,
        out_type=jax.ShapeDtypeStruct((B_TOTAL, D_MODEL), jnp.float32),
        scratch_types=[
            pltpu.VMEM((NJ, CH), jnp.int32),
            pltpu.VMEM((CH, 128), jnp.float32),
            pltpu.VMEM((CH, 128), jnp.float32),
            pltpu.VMEM((CH, D_MODEL), jnp.float32),
            pltpu.VMEM((CH, D_MODEL), jnp.float32),
            pltpu.SemaphoreType.DMA,
            pltpu.SemaphoreType.DMA,
            pltpu.SemaphoreType.DMA,
            pltpu.SemaphoreType.DMA,
        ],
        compiler_params=pltpu.CompilerParams(
            use_tc_tiling_on_sc=True, needs_layout_passes=False
        ),
    )(_emb_kernel)
    return kfn(xw, tabp)


def kernel(x, table):
    xw = x.astype(jnp.int32).reshape(NW, NJ, CH)
    tabp = jnp.pad(table, ((0, 0), (0, D_MODEL)))
    out = _embedding(xw, tabp)
    return out.reshape(x.shape[0], x.shape[1], D_MODEL)
